# baseline (speedup 1.0000x reference)
import functools

import jax
import jax.numpy as jnp
from jax import lax
from jax.experimental import pallas as pl
from jax.experimental.pallas import tpu as pltpu
from jax.experimental.pallas import tpu_sc as plsc

_B = 16384
_D = 64
_NC = 2
_NS = 16
_NW = _NC * _NS
_BPW = _B // _NW
_L = 16
_V = 1000000
_CR = 256
_NCH = (_V + _CR - 1) // _CR
_LAST = _V - (_NCH - 1) * _CR
_TPW = (_NCH + _NW - 1) // _NW


def _scalar(vec, k=0):
    return jnp.squeeze(lax.slice(vec, (k,), (k + 1,)))


def _sget(ref, j):
    return _scalar(plsc.load_gather(ref, [jnp.full((_L,), j, jnp.int32)]))


def _extract_body(sq_h, qp_h, csq_h, si_h, ip_h, csi_h, eq_hbm, ei_hbm,
                  oq_hbm, oi_hbm,
                  sq_v, qp_v, csq_v, si_v, ip_v, csi_v,
                  cbuf_v, cbuf2_v, tail_v, rowb_v, sem_c, sem_c2, sem_o):
    wid = lax.axis_index("s") * _NC + lax.axis_index("c")

    pltpu.sync_copy(sq_h, sq_v)
    pltpu.sync_copy(qp_h, qp_v)
    pltpu.sync_copy(csq_h, csq_v)
    pltpu.sync_copy(si_h, si_v)
    pltpu.sync_copy(ip_h, ip_v)
    pltpu.sync_copy(csi_h, csi_v)

    col0 = lax.iota(jnp.int32, _L)
    hi3 = jnp.right_shift(col0, 3)
    lo3 = jnp.bitwise_and(col0, 7)

    def extract_one(j, p, cbase, src_v, sv_v, pv_v, tbl_off, out_hbm):
        r = _sget(sv_v, j)
        b = _sget(pv_v, j)
        l = r - cbase
        lb = jnp.full((_L,), l, jnp.int32)
        slot = jnp.bitwise_and(j, 7) + tbl_off // 8
        for cc in range(_D // _L):
            vals = plsc.load_gather(src_v, [hi3 + 2 * cc, lo3, lb])
            rowb_v[slot, pl.ds(tbl_off + cc * _L, _L)] = vals
        pltpu.async_copy(rowb_v.at[slot], out_hbm.at[b], sem_o)
        @pl.when(p >= 8)
        def _():
            pltpu.make_async_copy(out_hbm.at[0], rowb_v.at[0], sem_o).wait()
        return jnp.minimum(p + 1, 8)

    def fetch_table(cid, eh, buf, sem):
        @pl.when(cid < _NCH - 1)
        def _():
            start = pl.multiple_of(cid * _CR, 128)
            pltpu.async_copy(eh.at[:, :, pl.ds(start, _CR)], buf, sem)

        @pl.when(cid == _NCH - 1)
        def _():
            pltpu.async_copy(eh.at[:, :, pl.ds((_NCH - 1) * _CR, _LAST)],
                             tail_v, sem)

    def wait_fetch(cid, eh, buf, sem):
        @pl.when(cid < _NCH - 1)
        def _():
            pltpu.make_async_copy(eh.at[:, :, pl.ds(0, _CR)], buf, sem).wait()

        @pl.when(cid == _NCH - 1)
        def _():
            pltpu.make_async_copy(eh.at[:, :, pl.ds((_NCH - 1) * _CR, _LAST)],
                                  tail_v, sem).wait()

    def do_table(cid, p, cbuf, sv_v, pv_v, cs_v, tbl_off, out_hbm):
        j0 = _sget(cs_v, cid)
        j1 = _sget(cs_v, cid + 1)
        p = lax.cond(
            cid < _NCH - 1,
            lambda pp: lax.fori_loop(
                j0, j1,
                lambda j, q: extract_one(j, q, cid * _CR, cbuf,
                                         sv_v, pv_v, tbl_off, out_hbm),
                pp),
            lambda pp: pp, p)
        p = lax.cond(
            cid == _NCH - 1,
            lambda pp: lax.fori_loop(
                j0, j1,
                lambda j, q: extract_one(j, q, (_NCH - 1) * _CR, tail_v,
                                         sv_v, pv_v, tbl_off, out_hbm),
                pp),
            lambda pp: pp, p)
        return p

    def trip(t, p):
        cid = wid + t * _NW
        def valid(pp):
            fetch_table(cid, eq_hbm, cbuf_v, sem_c)

            @pl.when(cid < _NCH - 1)
            def _():
                start = pl.multiple_of(cid * _CR, 128)
                pltpu.async_copy(ei_hbm.at[:, :, pl.ds(start, _CR)],
                                 cbuf2_v, sem_c2)

            wait_fetch(cid, eq_hbm, cbuf_v, sem_c)
            pp = do_table(cid, pp, cbuf_v, sq_v, qp_v, csq_v, 0, oq_hbm)

            @pl.when(cid == _NCH - 1)
            def _():
                pltpu.async_copy(ei_hbm.at[:, :, pl.ds((_NCH - 1) * _CR, _LAST)],
                                 tail_v, sem_c2)

            wait_fetch(cid, ei_hbm, cbuf2_v, sem_c2)
            pp = do_table(cid, pp, cbuf2_v, si_v, ip_v, csi_v, _D, oi_hbm)
            return pp
        return lax.cond(cid < _NCH, valid, lambda pp: pp, p)

    p = lax.fori_loop(0, _TPW, trip, jnp.int32(0))
    lax.fori_loop(0, p, lambda _, c: (
        pltpu.make_async_copy(oq_hbm.at[0], rowb_v.at[0], sem_o).wait(), c)[1], 0)


def _dot_body(oq_hbm, oi_hbm, out_hbm, viq_v, vii_v, part_v, scores_v, sem_q, sem_i):
    wid = lax.axis_index("s") * _NC + lax.axis_index("c")
    base = wid * _BPW

    col0 = lax.iota(jnp.int32, _L) * _L
    n_sub = 4
    sub = _BPW // n_sub

    for scn in range(n_sub):
        b0 = base + scn * sub
        cq = pltpu.async_copy(oq_hbm.at[pl.ds(b0, sub)], viq_v, sem_q)
        ci = pltpu.async_copy(oi_hbm.at[pl.ds(b0, sub)], vii_v, sem_i)
        cq.wait()
        ci.wait()

        def group(g, carry):
            r0 = g * _L
            for k in range(_L):
                r = r0 + k
                acc = viq_v[r, pl.ds(0, _L)] * vii_v[r, pl.ds(_D, _L)]
                for c in range(1, _D // _L):
                    acc = acc + (viq_v[r, pl.ds(c * _L, _L)]
                                 * vii_v[r, pl.ds(_D + c * _L, _L)])
                part_v[pl.ds(k * _L, _L)] = acc
            scores = plsc.load_gather(part_v, [col0])
            for l in range(1, _L):
                scores = scores + plsc.load_gather(part_v, [col0 + l])
            scores_v[pl.ds(scn * sub + r0, _L)] = 1.0 / (1.0 + jnp.exp(-scores))
            return carry

        lax.fori_loop(0, sub // _L, group, 0)

    pltpu.sync_copy(scores_v, out_hbm.at[pl.ds(base, _BPW)])


@jax.jit
def kernel(querys, items, embedding_query, embedding_item):
    eqT = jnp.transpose(embedding_query).reshape(8, 8, _V)
    eiT = jnp.transpose(embedding_item).reshape(8, 8, _V)

    qp = jnp.argsort(querys).astype(jnp.int32)
    sq = jnp.take(querys, qp)
    ip = jnp.argsort(items).astype(jnp.int32)
    si = jnp.take(items, ip)
    bounds = (jnp.arange(_NCH + 1, dtype=jnp.int32) * _CR).astype(jnp.int32)
    csq = jnp.searchsorted(sq, bounds, side="left", method="sort").astype(jnp.int32)
    csi = jnp.searchsorted(si, bounds, side="left", method="sort").astype(jnp.int32)
    pad = (-(_NCH + 1)) % 8
    csq = jnp.concatenate([csq, jnp.full((pad,), _B, jnp.int32)])
    csi = jnp.concatenate([csi, jnp.full((pad,), _B, jnp.int32)])

    mesh = plsc.VectorSubcoreMesh(
        core_axis_name="c", subcore_axis_name="s",
        num_cores=_NC, num_subcores=_NS)
    params = pltpu.CompilerParams(needs_layout_passes=False)

    k1 = functools.partial(
        pl.kernel,
        out_type=(jax.ShapeDtypeStruct((_B, 2 * _D), jnp.float32),
                  jax.ShapeDtypeStruct((_B, 2 * _D), jnp.float32)),
        mesh=mesh,
        compiler_params=params,
        scratch_types=[
            pltpu.VMEM((_B,), jnp.int32),
            pltpu.VMEM((_B,), jnp.int32),
            pltpu.VMEM((csq.shape[0],), jnp.int32),
            pltpu.VMEM((_B,), jnp.int32),
            pltpu.VMEM((_B,), jnp.int32),
            pltpu.VMEM((csq.shape[0],), jnp.int32),
            pltpu.VMEM((8, 8, _CR), jnp.float32),
            pltpu.VMEM((8, 8, _CR), jnp.float32),
            pltpu.VMEM((8, 8, _LAST), jnp.float32),
            pltpu.VMEM((16, 2 * _D), jnp.float32),
            pltpu.SemaphoreType.DMA,
            pltpu.SemaphoreType.DMA,
            pltpu.SemaphoreType.DMA,
        ],
    )(_extract_body)
    oq, oi = k1(sq, qp, csq, si, ip, csi, eqT, eiT)

    k2 = functools.partial(
        pl.kernel,
        out_type=jax.ShapeDtypeStruct((_B,), jnp.float32),
        mesh=mesh,
        compiler_params=params,
        scratch_types=[
            pltpu.VMEM((_BPW // 4, 2 * _D), jnp.float32),
            pltpu.VMEM((_BPW // 4, 2 * _D), jnp.float32),
            pltpu.VMEM((_L * _L,), jnp.float32),
            pltpu.VMEM((_BPW,), jnp.float32),
            pltpu.SemaphoreType.DMA,
            pltpu.SemaphoreType.DMA,
        ],
    )(_dot_body)
    return k2(oq, oi)


# speedup vs baseline: 1.3420x; 1.3420x over previous
import functools

import jax
import jax.numpy as jnp
from jax import lax
from jax.experimental import pallas as pl
from jax.experimental.pallas import tpu as pltpu
from jax.experimental.pallas import tpu_sc as plsc

_B = 16384
_D = 64
_NC = 2
_NS = 16
_NW = _NC * _NS
_BPW = _B // _NW
_L = 16
_CH = 32
_NCHUNK = _BPW // _CH


def _scalar(vec, k):
    return jnp.squeeze(lax.slice(vec, (k,), (k + 1,)))


def _mf_body(q_hbm, i_hbm, eq_hbm, ei_hbm, out_hbm,
             qidx_v, iidx_v, qblk_v, iblk_v, part_v, scores_v, sem_q, sem_i):
    wid = lax.axis_index("s") * _NC + lax.axis_index("c")
    base = wid * _BPW

    pltpu.sync_copy(q_hbm.at[pl.ds(base, _BPW)], qidx_v)
    pltpu.sync_copy(i_hbm.at[pl.ds(base, _BPW)], iidx_v)

    col0 = lax.iota(jnp.int32, _L) * _L

    def chunk(c, carry):
        r0 = c * _CH
        copies = []
        for g in range(_CH // _L):
            qv = qidx_v[pl.ds(r0 + g * _L, _L)]
            iv = iidx_v[pl.ds(r0 + g * _L, _L)]
            qb_vec = jnp.right_shift(qv, 3)
            ib_vec = jnp.right_shift(iv, 3)
            qs_vec = jnp.bitwise_and(qv, 7)
            is_vec = jnp.bitwise_and(iv, 7)
            for k in range(_L):
                kk = g * _L + k
                qb = _scalar(qb_vec, k)
                ib = _scalar(ib_vec, k)
                qs = _scalar(qs_vec, k)
                us = _scalar(is_vec, k)
                copies.append(pltpu.async_copy(eq_hbm.at[qb, qs], qblk_v.at[kk], sem_q))
                copies.append(pltpu.async_copy(ei_hbm.at[ib, us], iblk_v.at[kk], sem_i))
        for cp in copies:
            cp.wait()
        for g in range(_CH // _L):
            for k in range(_L):
                kk = g * _L + k
                acc = qblk_v[kk, pl.ds(0, _L)] * iblk_v[kk, pl.ds(0, _L)]
                for cc in range(1, _D // _L):
                    acc = acc + (qblk_v[kk, pl.ds(cc * _L, _L)]
                                 * iblk_v[kk, pl.ds(cc * _L, _L)])
                part_v[pl.ds(k * _L, _L)] = acc
            scores = plsc.load_gather(part_v, [col0])
            for l in range(1, _L):
                scores = scores + plsc.load_gather(part_v, [col0 + l])
            scores_v[pl.ds(r0 + g * _L, _L)] = 1.0 / (1.0 + jnp.exp(-scores))
        return carry

    lax.fori_loop(0, _NCHUNK, chunk, 0)

    pltpu.sync_copy(scores_v, out_hbm.at[pl.ds(base, _BPW)])


@jax.jit
def kernel(querys, items, embedding_query, embedding_item):
    eq3 = embedding_query.reshape(1000000 // 8, 8, _D)
    ei3 = embedding_item.reshape(1000000 // 8, 8, _D)
    mesh = plsc.VectorSubcoreMesh(
        core_axis_name="c", subcore_axis_name="s",
        num_cores=_NC, num_subcores=_NS)
    k = functools.partial(
        pl.kernel,
        out_type=jax.ShapeDtypeStruct((_B,), jnp.float32),
        mesh=mesh,
        compiler_params=pltpu.CompilerParams(needs_layout_passes=False),
        scratch_types=[
            pltpu.VMEM((_BPW,), jnp.int32),
            pltpu.VMEM((_BPW,), jnp.int32),
            pltpu.VMEM((_CH, _D), jnp.float32),
            pltpu.VMEM((_CH, _D), jnp.float32),
            pltpu.VMEM((_L * _L,), jnp.float32),
            pltpu.VMEM((_BPW,), jnp.float32),
            pltpu.SemaphoreType.DMA,
            pltpu.SemaphoreType.DMA,
        ],
    )(_mf_body)
    return k(querys, items, eq3, ei3)
